# Initial kernel scaffold; baseline (speedup 1.0000x reference)
import jax
import jax.numpy as jnp
from jax.experimental import pallas as pl


def kernel(bins, weights, n_samples):
    raise NotImplementedError("write your pallas kernel here")


# speedup vs baseline: 26.2693x; 26.2693x over previous
import functools

import jax
import jax.numpy as jnp
from jax import lax
from jax.experimental import pallas as pl
from jax.experimental.pallas import tpu as pltpu
from jax.experimental.pallas import tpu_sc as plsc

NC, NS, L = 2, 16, 16
NW = NC * NS
B, C = 65536, 128
RAYS_PER_W = B // NW
NT = RAYS_PER_W // L


def _sc_sample_pdf(bins_flat, w_flat):
    mesh = plsc.VectorSubcoreMesh(core_axis_name="c", subcore_axis_name="s")

    @functools.partial(
        pl.kernel,
        out_type=jax.ShapeDtypeStruct((B * C,), jnp.float32),
        mesh=mesh,
        compiler_params=pltpu.CompilerParams(needs_layout_passes=False),
        scratch_types=[
            pltpu.VMEM((L * (C - 1),), jnp.float32),
            pltpu.VMEM((L * C,), jnp.float32),
            pltpu.VMEM((C * L,), jnp.float32),
            pltpu.VMEM((C * L,), jnp.int32),
            pltpu.VMEM((L * C,), jnp.float32),
        ],
    )
    def k(bins_hbm, w_hbm, out_hbm, Wv, Bv, CDFv, HISTv, OUTv):
        wid = lax.axis_index("s") * NC + lax.axis_index("c")
        rows = lax.iota(jnp.int32, L)
        wrow = rows * (C - 1)
        brow = rows * C
        zf = jnp.zeros((L,), jnp.float32)
        zi = jnp.zeros((L,), jnp.int32)
        ones_i = jnp.ones((L,), jnp.int32)

        def zero_hist(i, carry):
            HISTv[pl.ds(i * L, L)] = zi
            return carry

        lax.fori_loop(0, C, zero_hist, 0)

        def tile_body(t, carry):
            base = wid * RAYS_PER_W + t * L
            pltpu.sync_copy(w_hbm.at[pl.ds(base * (C - 1), L * (C - 1))], Wv)
            pltpu.sync_copy(bins_hbm.at[pl.ds(base * C, L * C)], Bv)

            CDFv[pl.ds(0, L)] = zf

            def p1(j, acc):
                w = plsc.load_gather(Wv, [wrow + j])
                acc = acc + (w + 1e-5)
                CDFv[pl.ds((j + 1) * L, L)] = acc
                return acc

            total = lax.fori_loop(0, C - 1, p1, zf)
            inv_t = 1.0 / total
            c1 = inv_t * jnp.float32(C)

            def p2(j, carry):
                s = CDFv[pl.ds(j * L, L)]
                y = jnp.maximum(s * c1 - 0.5, 0.0)
                ki = y.astype(jnp.int32)
                kf = ki.astype(jnp.float32)
                ki = ki + (kf < y).astype(jnp.int32)
                msk = ki < C
                ki = jnp.minimum(ki, C - 1)
                plsc.addupdate_scatter(HISTv, [(ki << 4) + rows], ones_i, mask=msk)
                return carry

            lax.fori_loop(1, C, p2, 0)

            def p3(i, cnt):
                h = HISTv[pl.ds(i * L, L)]
                HISTv[pl.ds(i * L, L)] = zi
                below = cnt + h
                abv = jnp.minimum(below + 1, C - 1)
                s_b = plsc.load_gather(CDFv, [(below << 4) + rows])
                s_a = plsc.load_gather(CDFv, [(abv << 4) + rows])
                b_b = plsc.load_gather(Bv, [brow + below])
                b_a = plsc.load_gather(Bv, [brow + abv])
                u = (i.astype(jnp.float32) + 0.5) * jnp.float32(1.0 / 128.0)
                cdf_b = s_b * inv_t
                den = (s_a - s_b) * inv_t
                den = jnp.where(den < 1e-5, jnp.float32(1.0), den)
                t_frac = (u - cdf_b) / den
                res = b_b + t_frac * (b_a - b_b)
                plsc.store_scatter(OUTv, [brow + i], res)
                return below

            lax.fori_loop(0, C, p3, zi)
            pltpu.sync_copy(OUTv, out_hbm.at[pl.ds(base * C, L * C)])
            return carry

        lax.fori_loop(0, NT, tile_body, 0)

    return k(bins_flat, w_flat)


def kernel(bins, weights, n_samples):
    del n_samples
    out = _sc_sample_pdf(bins.reshape(-1), weights.reshape(-1))
    return out.reshape(B, C)


# speedup vs baseline: 32.8418x; 1.2502x over previous
import functools

import jax
import jax.numpy as jnp
from jax import lax
from jax.experimental import pallas as pl
from jax.experimental.pallas import tpu as pltpu
from jax.experimental.pallas import tpu_sc as plsc

NC, NS, L = 2, 16, 16
NW = NC * NS
B, C = 65536, 128
RAYS_PER_W = B // NW
NT = RAYS_PER_W // L
HR = C + 2
UN = 8


def _sc_sample_pdf(bins_flat, w_flat):
    mesh = plsc.VectorSubcoreMesh(core_axis_name="c", subcore_axis_name="s")

    @functools.partial(
        pl.kernel,
        out_type=jax.ShapeDtypeStruct((B * C,), jnp.float32),
        mesh=mesh,
        compiler_params=pltpu.CompilerParams(needs_layout_passes=False),
        scratch_types=[
            pltpu.VMEM((L * (C - 1),), jnp.float32),
            pltpu.VMEM((L * (C - 1),), jnp.float32),
            pltpu.VMEM((L * C,), jnp.float32),
            pltpu.VMEM((L * C,), jnp.float32),
            pltpu.VMEM((L * C,), jnp.float32),
            pltpu.VMEM((L * C,), jnp.float32),
            pltpu.VMEM((C * L,), jnp.float32),
            pltpu.VMEM((HR * L,), jnp.int32),
            pltpu.SemaphoreType.DMA,
            pltpu.SemaphoreType.DMA,
            pltpu.SemaphoreType.DMA,
            pltpu.SemaphoreType.DMA,
            pltpu.SemaphoreType.DMA,
            pltpu.SemaphoreType.DMA,
        ],
    )
    def k(bins_hbm, w_hbm, out_hbm, Wv0, Wv1, Bv0, Bv1, Ov0, Ov1, CDFv,
          HISTv, ws0, ws1, bs0, bs1, os0, os1):
        wid = lax.axis_index("s") * NC + lax.axis_index("c")
        rows = lax.iota(jnp.int32, L)
        wrow = rows * (C - 1)
        brow = rows * C
        zf = jnp.zeros((L,), jnp.float32)
        zi = jnp.zeros((L,), jnp.int32)
        ones_i = jnp.ones((L,), jnp.int32)
        bufs = ((Wv0, Bv0, Ov0, ws0, bs0, os0), (Wv1, Bv1, Ov1, ws1, bs1, os1))
        wbase0 = wid * RAYS_PER_W * (C - 1)
        bbase0 = wid * RAYS_PER_W * C

        def zero_hist(i, carry):
            HISTv[pl.ds(i * L, L)] = zi
            return carry

        lax.fori_loop(0, HR, zero_hist, 0, unroll=UN)

        for b, (Wv, Bv, _, wsem, bsem, _) in enumerate(bufs):
            pltpu.async_copy(
                w_hbm.at[pl.ds(wbase0 + b * L * (C - 1), L * (C - 1))], Wv, wsem)
            pltpu.async_copy(
                bins_hbm.at[pl.ds(bbase0 + b * L * C, L * C)], Bv, bsem)

        def pair_body(g, carry):
            for b, (Wv, Bv, Ov, wsem, bsem, osem) in enumerate(bufs):
                t = 2 * g + b
                woff = wbase0 + t * L * (C - 1)
                boff = bbase0 + t * L * C

                pltpu.make_async_copy(
                    w_hbm.at[pl.ds(woff, L * (C - 1))], Wv, wsem).wait()

                CDFv[pl.ds(0, L)] = zf

                def p1(j, acc):
                    w = plsc.load_gather(Wv, [wrow + j])
                    acc = acc + (w + 1e-5)
                    CDFv[pl.ds((j + 1) * L, L)] = acc
                    return acc

                total = lax.fori_loop(0, C - 1, p1, zf, unroll=UN)
                inv_t = 1.0 / total
                c1 = inv_t * jnp.float32(C)

                @pl.when(t + 2 < NT)
                def _():
                    pltpu.async_copy(
                        w_hbm.at[pl.ds(woff + 2 * L * (C - 1), L * (C - 1))],
                        Wv, wsem)

                def p2(j, carry2):
                    s = CDFv[pl.ds(j * L, L)]
                    y = s * c1 - 0.5
                    ki = y.astype(jnp.int32)
                    kf = ki.astype(jnp.float32)
                    ki = ki + (kf < y).astype(jnp.int32)
                    plsc.addupdate_scatter(HISTv, [(ki << 4) + rows], ones_i)
                    return carry2

                lax.fori_loop(1, C, p2, 0, unroll=UN)

                pltpu.make_async_copy(
                    bins_hbm.at[pl.ds(boff, L * C)], Bv, bsem).wait()

                @pl.when(g > 0)
                def _():
                    pltpu.make_async_copy(
                        Ov, out_hbm.at[pl.ds(boff, L * C)], osem).wait()

                def p3(i, cnt):
                    h = HISTv[pl.ds(i * L, L)]
                    HISTv[pl.ds(i * L, L)] = zi
                    below = cnt + h
                    abv = jnp.minimum(below + 1, C - 1)
                    s_b = plsc.load_gather(CDFv, [(below << 4) + rows])
                    s_a = plsc.load_gather(CDFv, [(abv << 4) + rows])
                    b_b = plsc.load_gather(Bv, [brow + below])
                    b_a = plsc.load_gather(Bv, [brow + abv])
                    u = (i.astype(jnp.float32) + 0.5) * jnp.float32(1.0 / 128.0)
                    cdf_b = s_b * inv_t
                    den = (s_a - s_b) * inv_t
                    den = jnp.where(den < 1e-5, jnp.float32(1.0), den)
                    t_frac = (u - cdf_b) / den
                    res = b_b + t_frac * (b_a - b_b)
                    plsc.store_scatter(Ov, [brow + i], res)
                    return below

                lax.fori_loop(0, C, p3, zi, unroll=UN)

                @pl.when(t + 2 < NT)
                def _():
                    pltpu.async_copy(
                        bins_hbm.at[pl.ds(boff + 2 * L * C, L * C)], Bv, bsem)

                pltpu.async_copy(Ov, out_hbm.at[pl.ds(boff, L * C)], osem)
            return carry

        lax.fori_loop(0, NT // 2, pair_body, 0)

        for b, (_, _, Ov, _, _, osem) in enumerate(bufs):
            off = bbase0 + (NT - 2 + b) * L * C
            pltpu.make_async_copy(Ov, out_hbm.at[pl.ds(off, L * C)], osem).wait()

    return k(bins_flat, w_flat)


def kernel(bins, weights, n_samples):
    del n_samples
    out = _sc_sample_pdf(bins.reshape(-1), weights.reshape(-1))
    return out.reshape(B, C)
